# baseline (speedup 1.0000x reference)
import functools

import jax
import jax.numpy as jnp
from jax.experimental import pallas as pl

_EPS_T = 1e-5
_NEG_BIG = -3.0e38
_TILE_V = 2048
_ROWS = 16


def _logits_kernel(hs_ref, emb_ref, t_ref, out_ref):
    acc = jax.lax.dot_general(
        hs_ref[...], emb_ref[...],
        (((1,), (1,)), ((), ())),
        preferred_element_type=jnp.float32)
    acc = jnp.where(jnp.isfinite(acc), acc, 0.0)
    out_ref[...] = acc / t_ref[...]


def _select_kernel(x_ref, g_ref, temp_ref, p_ref, k_ref, out_ref, *, vocab):
    x = x_ref[...]
    g = g_ref[...]
    temp = temp_ref[...]
    p = p_ref[...]
    k = jnp.clip(k_ref[...], 1, vocab)

    rows = x.shape[0]
    half = rows // 2
    ubits = jax.lax.bitcast_convert_type(x, jnp.uint32)
    sign = jnp.uint32(0x80000000)
    s = jnp.where(ubits >= sign, ~ubits, ubits | sign)

    one = jnp.uint32(1)
    sa, sb = s[:half], s[half:]
    ka, kb = k[:half], k[half:]

    zero_t = jnp.zeros((half, 1), jnp.uint32)
    tha, thb = zero_t, zero_t
    for bitpos in range(31, -1, -1):
        bit = jnp.uint32(1 << bitpos)
        t2a = tha | bit
        t2b = thb | bit
        ca = jnp.count_nonzero(sa >= t2a, axis=1, keepdims=True)
        cb = jnp.count_nonzero(sb >= t2b, axis=1, keepdims=True)
        tha = jnp.where(ca >= ka, t2a, tha)
        thb = jnp.where(cb >= kb, t2b, thb)
    thresh = jnp.concatenate([tha, thb], axis=0)
    apply_k = k < vocab
    keep_k = (s >= thresh) | (~apply_k)

    m = jnp.max(x, axis=1, keepdims=True)
    e = jnp.where(keep_k, jnp.exp(x - m), 0.0)
    z1 = jnp.sum(e, axis=1, keepdims=True)
    pz = p * z1
    ea, eb = e[:half], e[half:]
    pza, pzb = pz[:half], pz[half:]

    th2a, th2b = zero_t, zero_t
    for bitpos in range(31, -1, -1):
        bit = jnp.uint32(1 << bitpos)
        t2a = th2a | bit
        t2b = th2b | bit
        fa = jnp.sum(jnp.where(sa >= t2a, ea, 0.0), axis=1, keepdims=True)
        fb = jnp.sum(jnp.where(sb >= t2b, eb, 0.0), axis=1, keepdims=True)
        th2a = jnp.where(fa > pza, t2a, th2a)
        th2b = jnp.where(fb > pzb, t2b, th2b)
    c0 = jnp.concatenate([th2a, th2b], axis=0) + one
    si = jax.lax.bitcast_convert_type(s ^ sign, jnp.int32)
    imax = jnp.int32(0x7FFFFFFF)
    cand = jnp.where(keep_k & (s >= c0), si, imax)
    cmin = jnp.min(cand, axis=1, keepdims=True)
    code_m = jnp.max(si, axis=1, keepdims=True)
    cutoff = jnp.where(cmin == imax, code_m, cmin)
    apply_p = p < (1.0 - _EPS_T)
    keep = keep_k & (jnp.logical_not(apply_p) | (si >= cutoff))

    z3 = jnp.sum(jnp.where(keep, e, 0.0), axis=1, keepdims=True)
    score = jnp.where(keep, jnp.log(e / z3 + 1e-20) + g, _NEG_BIG)

    col = jax.lax.broadcasted_iota(jnp.int32, x.shape, 1)
    smax = jnp.max(score, axis=1, keepdims=True)
    sampled = jnp.min(jnp.where(score == smax, col, vocab), axis=1)
    greedy = jnp.min(jnp.where(si == code_m, col, vocab), axis=1)
    token = jnp.where(temp[:, 0] < _EPS_T, greedy, sampled)
    token = jnp.where((token < 0) | (token >= vocab), 0, token)
    out_ref[...] = token[:, None]


def kernel(hidden_states, embedding, last_token_indices, temperatures,
           top_ps, top_ks):
    n_rows = hidden_states.shape[0]
    vocab, dim = embedding.shape

    hs = jnp.take(hidden_states, last_token_indices, axis=0)
    t = jnp.where(temperatures < _EPS_T, 1.0, temperatures)
    t = t.astype(jnp.float32).reshape(n_rows, 1)

    logits = pl.pallas_call(
        _logits_kernel,
        grid=(pl.cdiv(vocab, _TILE_V),),
        in_specs=[
            pl.BlockSpec((n_rows, dim), lambda i: (0, 0)),
            pl.BlockSpec((_TILE_V, dim), lambda i: (i, 0)),
            pl.BlockSpec((n_rows, 1), lambda i: (0, 0)),
        ],
        out_specs=pl.BlockSpec((n_rows, _TILE_V), lambda i: (0, i)),
        out_shape=jax.ShapeDtypeStruct((n_rows, vocab), jnp.float32),
    )(hs, embedding, t)

    gumbel = jax.random.gumbel(
        jax.random.key(42), (n_rows, vocab), jnp.float32)

    rows = _ROWS if n_rows % _ROWS == 0 else n_rows
    tokens = pl.pallas_call(
        functools.partial(_select_kernel, vocab=vocab),
        grid=(n_rows // rows,),
        in_specs=[
            pl.BlockSpec((rows, vocab), lambda i: (i, 0)),
            pl.BlockSpec((rows, vocab), lambda i: (i, 0)),
            pl.BlockSpec((rows, 1), lambda i: (i, 0)),
            pl.BlockSpec((rows, 1), lambda i: (i, 0)),
            pl.BlockSpec((rows, 1), lambda i: (i, 0)),
        ],
        out_specs=pl.BlockSpec((rows, 1), lambda i: (i, 0)),
        out_shape=jax.ShapeDtypeStruct((n_rows, 1), jnp.int32),
    )(logits, gumbel,
      temperatures.astype(jnp.float32).reshape(n_rows, 1),
      top_ps.astype(jnp.float32).reshape(n_rows, 1),
      top_ks.astype(jnp.int32).reshape(n_rows, 1))

    return tokens.reshape(n_rows)


# speedup vs baseline: 1.0766x; 1.0766x over previous
import functools

import jax
import jax.numpy as jnp
from jax.experimental import pallas as pl

_EPS_T = 1e-5
_NEG_BIG = -3.0e38
_TILE_V = 4096
_ROWS = 16


def _logits_kernel(hs_ref, emb_ref, t_ref, out_ref):
    acc = jax.lax.dot_general(
        hs_ref[...], emb_ref[...],
        (((1,), (1,)), ((), ())),
        preferred_element_type=jnp.float32)
    acc = jnp.where(jnp.isfinite(acc), acc, 0.0)
    out_ref[...] = acc / t_ref[...]


def _select_kernel(x_ref, g_ref, temp_ref, p_ref, k_ref, out_ref, *, vocab):
    x = x_ref[...]
    g = g_ref[...]
    temp = temp_ref[...]
    p = p_ref[...]
    k = jnp.clip(k_ref[...], 1, vocab)

    rows = x.shape[0]
    gsz = 8
    ng = rows // gsz
    ubits = jax.lax.bitcast_convert_type(x, jnp.uint32)
    sign = jnp.uint32(0x80000000)
    s = jnp.where(ubits >= sign, ~ubits, ubits | sign)

    one = jnp.uint32(1)
    sg = [s[i * gsz:(i + 1) * gsz] for i in range(ng)]
    kg = [k[i * gsz:(i + 1) * gsz] for i in range(ng)]

    def kth_body(i, carry):
        bit = one << (31 - i).astype(jnp.uint32)
        out = []
        for gi in range(ng):
            t2 = carry[gi] | bit
            cnt = jnp.count_nonzero(sg[gi] >= t2, axis=1, keepdims=True)
            out.append(jnp.where(cnt >= kg[gi], t2, carry[gi]))
        return tuple(out)

    zero_t = jnp.zeros((gsz, 1), jnp.uint32)
    th_g = jax.lax.fori_loop(0, 32, kth_body, (zero_t,) * ng)
    thresh = jnp.concatenate(th_g, axis=0)
    apply_k = k < vocab
    keep_k = (s >= thresh) | (~apply_k)

    m = jnp.max(x, axis=1, keepdims=True)
    e = jnp.where(keep_k, jnp.exp(x - m), 0.0)
    z1 = jnp.sum(e, axis=1, keepdims=True)
    pz = p * z1
    eg = [e[i * gsz:(i + 1) * gsz] for i in range(ng)]
    pzg = [pz[i * gsz:(i + 1) * gsz] for i in range(ng)]

    def cut_body(i, carry):
        bit = one << (31 - i).astype(jnp.uint32)
        out = []
        for gi in range(ng):
            t2 = carry[gi] | bit
            f = jnp.sum(jnp.where(sg[gi] >= t2, eg[gi], 0.0),
                        axis=1, keepdims=True)
            out.append(jnp.where(f > pzg[gi], t2, carry[gi]))
        return tuple(out)

    th2_g = jax.lax.fori_loop(0, 32, cut_body, (zero_t,) * ng)
    c0 = jnp.concatenate(th2_g, axis=0) + one
    si = jax.lax.bitcast_convert_type(s ^ sign, jnp.int32)
    imax = jnp.int32(0x7FFFFFFF)
    cand = jnp.where(keep_k & (s >= c0), si, imax)
    cmin = jnp.min(cand, axis=1, keepdims=True)
    code_m = jnp.max(si, axis=1, keepdims=True)
    cutoff = jnp.where(cmin == imax, code_m, cmin)
    apply_p = p < (1.0 - _EPS_T)
    keep = keep_k & (jnp.logical_not(apply_p) | (si >= cutoff))

    z3 = jnp.sum(jnp.where(keep, e, 0.0), axis=1, keepdims=True)
    score = jnp.where(keep, jnp.log(e / z3 + 1e-20) + g, _NEG_BIG)

    col = jax.lax.broadcasted_iota(jnp.int32, x.shape, 1)
    smax = jnp.max(score, axis=1, keepdims=True)
    sampled = jnp.min(jnp.where(score == smax, col, vocab), axis=1)
    greedy = jnp.min(jnp.where(si == code_m, col, vocab), axis=1)
    token = jnp.where(temp[:, 0] < _EPS_T, greedy, sampled)
    token = jnp.where((token < 0) | (token >= vocab), 0, token)
    out_ref[...] = token[:, None]


def kernel(hidden_states, embedding, last_token_indices, temperatures,
           top_ps, top_ks):
    n_rows = hidden_states.shape[0]
    vocab, dim = embedding.shape

    hs = jnp.take(hidden_states, last_token_indices, axis=0)
    t = jnp.where(temperatures < _EPS_T, 1.0, temperatures)
    t = t.astype(jnp.float32).reshape(n_rows, 1)

    logits = pl.pallas_call(
        _logits_kernel,
        grid=(pl.cdiv(vocab, _TILE_V),),
        in_specs=[
            pl.BlockSpec((n_rows, dim), lambda i: (0, 0)),
            pl.BlockSpec((_TILE_V, dim), lambda i: (i, 0)),
            pl.BlockSpec((n_rows, 1), lambda i: (0, 0)),
        ],
        out_specs=pl.BlockSpec((n_rows, _TILE_V), lambda i: (0, i)),
        out_shape=jax.ShapeDtypeStruct((n_rows, vocab), jnp.float32),
    )(hs, embedding, t)

    gumbel = jax.random.gumbel(
        jax.random.key(42), (n_rows, vocab), jnp.float32)

    rows = _ROWS if n_rows % _ROWS == 0 else n_rows
    tokens = pl.pallas_call(
        functools.partial(_select_kernel, vocab=vocab),
        grid=(n_rows // rows,),
        in_specs=[
            pl.BlockSpec((rows, vocab), lambda i: (i, 0)),
            pl.BlockSpec((rows, vocab), lambda i: (i, 0)),
            pl.BlockSpec((rows, 1), lambda i: (i, 0)),
            pl.BlockSpec((rows, 1), lambda i: (i, 0)),
            pl.BlockSpec((rows, 1), lambda i: (i, 0)),
        ],
        out_specs=pl.BlockSpec((rows, 1), lambda i: (i, 0)),
        out_shape=jax.ShapeDtypeStruct((n_rows, 1), jnp.int32),
    )(logits, gumbel,
      temperatures.astype(jnp.float32).reshape(n_rows, 1),
      top_ps.astype(jnp.float32).reshape(n_rows, 1),
      top_ks.astype(jnp.int32).reshape(n_rows, 1))

    return tokens.reshape(n_rows)
